# baseline (speedup 1.0000x reference)
import functools

import jax
import jax.numpy as jnp
from jax import lax
from jax.experimental import pallas as pl
from jax.experimental.pallas import tpu as pltpu
from jax.experimental.pallas import tpu_sc as plsc

_NUM_CORES = 2
_NUM_SUBCORES = 16
_NUM_WORKERS = _NUM_CORES * _NUM_SUBCORES


def _pe_broadcast(pe, batch, seq_len, chunk):
    embed_dim = pe.shape[1]
    rows_per_w = seq_len // _NUM_WORKERS
    n_chunks = rows_per_w // chunk
    mesh = plsc.VectorSubcoreMesh(
        core_axis_name="c",
        subcore_axis_name="s",
        num_cores=_NUM_CORES,
        num_subcores=_NUM_SUBCORES,
    )

    del n_chunks

    @functools.partial(
        pl.kernel,
        out_type=jax.ShapeDtypeStruct((batch, seq_len, embed_dim), pe.dtype),
        mesh=mesh,
        scratch_types=[
            pltpu.SemaphoreType.DMA,
        ],
    )
    def broadcast_kernel(pe_hbm, out_hbm, sem):
        wid = lax.axis_index("s") * _NUM_CORES + lax.axis_index("c")
        row0 = wid * rows_per_w
        copies = [
            pltpu.async_copy(
                pe_hbm.at[pl.ds(row0, rows_per_w)],
                out_hbm.at[b, pl.ds(row0, rows_per_w)],
                sem,
            )
            for b in range(batch)
        ]
        for h in copies:
            h.wait()

    return broadcast_kernel


def kernel(x, pe):
    batch, seq_len = x.shape[0], x.shape[1]
    return _pe_broadcast(pe, batch, seq_len, chunk=32)(pe)


# speedup vs baseline: 22.3003x; 22.3003x over previous
import functools

import jax
import jax.numpy as jnp
from jax import lax
from jax.experimental import pallas as pl
from jax.experimental.pallas import tpu as pltpu
from jax.experimental.pallas import tpu_sc as plsc

_NUM_CORES = 2
_NUM_SUBCORES = 16
_NUM_WORKERS = _NUM_CORES * _NUM_SUBCORES


def _pe_broadcast_sc(pe, batch, seq_len, chunk):
    embed_dim = pe.shape[1]
    rows_per_w = seq_len // _NUM_WORKERS
    n_chunks = rows_per_w // chunk
    mesh = plsc.VectorSubcoreMesh(
        core_axis_name="c",
        subcore_axis_name="s",
        num_cores=_NUM_CORES,
        num_subcores=_NUM_SUBCORES,
    )

    @functools.partial(
        pl.kernel,
        out_type=jax.ShapeDtypeStruct((batch, seq_len, embed_dim), pe.dtype),
        mesh=mesh,
        scratch_types=[
            pltpu.VMEM((chunk, embed_dim), pe.dtype),
        ],
    )
    def broadcast_kernel(pe_hbm, out_hbm, buf):
        wid = lax.axis_index("s") * _NUM_CORES + lax.axis_index("c")
        row0 = wid * rows_per_w
        for c in range(n_chunks):
            base = row0 + c * chunk
            pltpu.sync_copy(pe_hbm.at[pl.ds(base, chunk)], buf)
            for b in range(batch):
                pltpu.sync_copy(buf, out_hbm.at[b, pl.ds(base, chunk)])

    return broadcast_kernel


def _pe_broadcast_tc(pe, batch, seq_len, block_s):
    embed_dim = pe.shape[1]
    n_blocks = seq_len // block_s

    def body(pe_ref, out_ref):
        out_ref[0] = pe_ref[...]

    return pl.pallas_call(
        body,
        grid=(n_blocks, batch),
        in_specs=[
            pl.BlockSpec((block_s, embed_dim), lambda s, b: (s, 0)),
        ],
        out_specs=pl.BlockSpec((1, block_s, embed_dim), lambda s, b: (b, s, 0)),
        out_shape=jax.ShapeDtypeStruct((batch, seq_len, embed_dim), pe.dtype),
    )


def kernel(x, pe):
    batch, seq_len = x.shape[0], x.shape[1]
    sc_batch = batch // 2
    tc_batch = batch - sc_batch
    out_sc = _pe_broadcast_sc(pe, sc_batch, seq_len, chunk=64)(pe)
    out_tc = _pe_broadcast_tc(pe, tc_batch, seq_len, block_s=512)(pe)
    return jnp.concatenate([out_tc, out_sc], axis=0)


# speedup vs baseline: 52.9508x; 2.3744x over previous
import jax
import jax.numpy as jnp
from jax.experimental import pallas as pl


def _pe_broadcast_tc(pe, batch, seq_len, block_s):
    embed_dim = pe.shape[1]
    n_blocks = seq_len // block_s

    def body(pe_ref, out_ref):
        out_ref[0] = pe_ref[...]

    return pl.pallas_call(
        body,
        grid=(n_blocks, batch),
        in_specs=[
            pl.BlockSpec((block_s, embed_dim), lambda s, b: (s, 0)),
        ],
        out_specs=pl.BlockSpec((1, block_s, embed_dim), lambda s, b: (b, s, 0)),
        out_shape=jax.ShapeDtypeStruct((batch, seq_len, embed_dim), pe.dtype),
    )


def kernel(x, pe):
    batch, seq_len = x.shape[0], x.shape[1]
    return _pe_broadcast_tc(pe, batch, seq_len, block_s=512)(pe)
